# Initial kernel scaffold; baseline (speedup 1.0000x reference)
import jax
import jax.numpy as jnp
from jax.experimental import pallas as pl


def kernel(x, edge_index, W, V):
    raise NotImplementedError("write your pallas kernel here")


# speedup vs baseline: 6.9493x; 6.9493x over previous
import functools

import jax
import jax.numpy as jnp
from jax import lax
from jax.experimental import pallas as pl
from jax.experimental.pallas import tpu as pltpu
from jax.experimental.pallas import tpu_sc as plsc

N = 10000
E = 320000
IN_FEA = 128
HIDDEN = 128
RANK = 64

NC = 2
NS = 16
NW = NC * NS

B = 128
K = -(-E // (NW * B))
EPAD = NW * K * B
NPAD = 10240
ROWS_PER_TILE = NPAD // NS

W2 = 2 * RANK


def _prep_body(x_ref, w_ref, p_ref):
    feat = jnp.dot(x_ref[...], w_ref[...], preferred_element_type=jnp.float32)
    logp = jnp.log(jnp.maximum(jnp.abs(feat), 1e-30))
    gp = jnp.where(feat < 0, 3.0, 2.0)
    p_ref[...] = jnp.concatenate([logp, gp], axis=1)


def _finish_body(pp_ref, v_ref, o_ref):
    a = pp_ref[0] + pp_ref[1]
    s = a[:, :RANK]
    g = a[:, RANK:]
    sign = 1.0 - 2.0 * jnp.mod(g, 2.0)
    neigh = jnp.where(g > 0.0, sign * jnp.exp(s), 0.0)
    o_ref[...] = lax.dot_general(neigh, v_ref[...],
                                 (((1,), (1,)), ((), ())),
                                 preferred_element_type=jnp.float32)


def _sc_scatter_body(p_hbm, src_hbm, dst_hbm, zeros_hbm, out_hbm,
                     src_v, dst_v, gbuf, acc, sem):
    c = lax.axis_index("c")
    s = lax.axis_index("s")
    wid = s * NC + c

    pltpu.sync_copy(zeros_hbm.at[pl.ds(s * ROWS_PER_TILE, ROWS_PER_TILE)],
                    acc.at[pl.ds(s * ROWS_PER_TILE, ROWS_PER_TILE)])
    pltpu.sync_copy(src_hbm.at[wid], src_v)
    pltpu.sync_copy(dst_hbm.at[wid], dst_v)
    plsc.subcore_barrier()

    def body(j, carry):
        pltpu.async_copy(p_hbm.at[src_v.at[j]], gbuf, sem).wait()
        pltpu.sync_copy(gbuf, acc.at[dst_v.at[j]], add=True)
        return carry

    lax.fori_loop(0, K, body, 0)
    plsc.subcore_barrier()

    pltpu.sync_copy(acc.at[pl.ds(s * ROWS_PER_TILE, ROWS_PER_TILE)],
                    out_hbm.at[c, pl.ds(s * ROWS_PER_TILE, ROWS_PER_TILE)])


_sc_scatter = functools.partial(
    pl.kernel,
    out_type=jax.ShapeDtypeStruct((NC, NPAD, W2), jnp.float32),
    mesh=plsc.VectorSubcoreMesh(core_axis_name="c", subcore_axis_name="s"),
    scratch_types=[
        pltpu.VMEM((K, B), jnp.int32),
        pltpu.VMEM((K, B), jnp.int32),
        pltpu.VMEM((B, W2), jnp.float32),
        pltpu.VMEM_SHARED((NPAD, W2), jnp.float32),
        pltpu.SemaphoreType.DMA,
    ],
)(_sc_scatter_body)


def kernel(x, edge_index, W, V):
    xp = jnp.concatenate(
        [x, jnp.zeros((NPAD - N, IN_FEA), jnp.float32)], axis=0)

    blk = 1024
    P = pl.pallas_call(
        _prep_body,
        grid=(NPAD // blk,),
        in_specs=[
            pl.BlockSpec((blk, IN_FEA), lambda i: (i, 0)),
            pl.BlockSpec((IN_FEA, RANK), lambda i: (0, 0)),
        ],
        out_specs=pl.BlockSpec((blk, W2), lambda i: (i, 0)),
        out_shape=jax.ShapeDtypeStruct((NPAD, W2), jnp.float32),
    )(xp, W)

    pad = EPAD - E
    src = jnp.concatenate([edge_index[0], jnp.zeros((pad,), jnp.int32)])
    dst = jnp.concatenate([edge_index[1], jnp.full((pad,), N, jnp.int32)])
    src_r = src.reshape(NW, K, B)
    dst_r = dst.reshape(NW, K, B)
    zeros = jnp.zeros((NPAD, W2), jnp.float32)

    partials = _sc_scatter(P, src_r, dst_r, zeros)

    blk2 = 1000
    out = pl.pallas_call(
        _finish_body,
        grid=(N // blk2,),
        in_specs=[
            pl.BlockSpec((NC, blk2, W2), lambda i: (0, i, 0)),
            pl.BlockSpec((IN_FEA, RANK), lambda i: (0, 0)),
        ],
        out_specs=pl.BlockSpec((blk2, HIDDEN), lambda i: (i, 0)),
        out_shape=jax.ShapeDtypeStruct((N, HIDDEN), jnp.float32),
    )(partials, V)
    return out
